# baseline (speedup 1.0000x reference)
import functools

import jax
import jax.numpy as jnp
from jax import lax
from jax.experimental import pallas as pl
from jax.experimental.pallas import tpu as pltpu
from jax.experimental.pallas import tpu_sc as plsc

ROWS = 16384
COLS = 26
CARD = 100
OUT_W = COLS * CARD

NW = 32
ROWS_PER_W = ROWS // NW
CB = 200
NCB = OUT_W // CB
BR = 256
NB = NCB * 2
XW = ROWS_PER_W * COLS


def _onehot_body(xt_hbm, out_hbm, xv, buf0, buf1, sem0, sem1, semx):
    wid = lax.axis_index("s") * 2 + lax.axis_index("c")
    iota = lax.iota(jnp.int32, 16)
    ones = jnp.full((16,), 1, jnp.int32)
    zeros = jnp.zeros((16,), jnp.int32)
    row0 = wid * ROWS_PER_W

    pltpu.async_copy(xt_hbm.at[:, pl.ds(row0, ROWS_PER_W)], xv, semx)

    def make_zbody(buf):
        def zbody(k, carry):
            rvec = k * 16 + iota
            for c in range(0, CB, 2):
                plsc.store_scatter(buf, [jnp.full((16,), c, jnp.int32), rvec], zeros)
                plsc.store_scatter(buf, [jnp.full((16,), c + 1, jnp.int32), rvec], zeros)
            return carry
        return zbody

    lax.fori_loop(0, BR // 16, make_zbody(buf0), 0)
    pltpu.make_async_copy(xt_hbm.at[:, pl.ds(row0, ROWS_PER_W)], xv, semx).wait()

    def scatter(buf, q, value_vec):
        cb = q // 2
        hh = q % 2
        for half in range(2):
            c = 2 * cb + half
            cvec = jnp.full((16,), 1, jnp.int32) * c
            for k in range(BR // 16):
                ridx = hh * BR + k * 16 + iota
                vals = plsc.load_gather(xv, [cvec, ridx])
                col = vals + half * CARD
                plsc.store_scatter(buf, [col, k * 16 + iota], value_vec)

    def out_slice(q):
        cb = q // 2
        hh = q % 2
        return out_hbm.at[pl.ds(cb * CB, CB), pl.ds(row0 + hh * BR, BR)]

    scatter(buf0, 0, ones)
    pltpu.async_copy(buf0, out_slice(0), sem0)
    lax.fori_loop(0, BR // 16, make_zbody(buf1), 0)
    scatter(buf1, 1, ones)
    pltpu.async_copy(buf1, out_slice(1), sem1)

    def tbody(t, carry):
        for b, (buf, sem) in enumerate(((buf0, sem0), (buf1, sem1))):
            q = 2 * t + b
            pltpu.make_async_copy(buf, out_slice(q - 2), sem).wait()
            scatter(buf, q - 2, zeros)
            scatter(buf, q, ones)
            pltpu.async_copy(buf, out_slice(q), sem)
        return carry
    lax.fori_loop(1, NB // 2, tbody, 0)

    pltpu.make_async_copy(buf0, out_slice(NB - 2), sem0).wait()
    pltpu.make_async_copy(buf1, out_slice(NB - 1), sem1).wait()


@jax.jit
def kernel(x):
    mesh = plsc.VectorSubcoreMesh(core_axis_name="c", subcore_axis_name="s")
    run = functools.partial(
        pl.kernel,
        mesh=mesh,
        out_type=jax.ShapeDtypeStruct((OUT_W, ROWS), jnp.int32),
        scratch_types=[
            pltpu.VMEM((COLS, ROWS_PER_W), jnp.int32),
            pltpu.VMEM((CB, BR), jnp.int32),
            pltpu.VMEM((CB, BR), jnp.int32),
            pltpu.SemaphoreType.DMA,
            pltpu.SemaphoreType.DMA,
            pltpu.SemaphoreType.DMA,
        ],
        compiler_params=pltpu.CompilerParams(needs_layout_passes=False),
    )(_onehot_body)
    out_t = run(x.T)
    return out_t.T
